# Initial kernel scaffold; baseline (speedup 1.0000x reference)
import jax
import jax.numpy as jnp
from jax.experimental import pallas as pl


def kernel(x, hg, pd, Wt1, bt1, Wt2, bt2, Wg0, bg0, Wl0, bl0, Wg1, bg1, Wl1, bl1):
    raise NotImplementedError("write your pallas kernel here")


# speedup vs baseline: 4.5676x; 4.5676x over previous
import dataclasses
import functools

import jax
import jax.numpy as jnp
from jax import lax
from jax.experimental import pallas as pl
from jax.experimental.pallas import tpu as pltpu
from jax.experimental.pallas import tpu_sc as plsc

N = 10000
E = 320000
D = 128
CNT_W = 16
NC = 2
NS = 16
NW = NC * NS
PAIRS_PER_TILE = E // NW
CHUNK = 80
N_ITERS = PAIRS_PER_TILE // CHUNK
N_PAD = 10240
ROWS_PER_TILE = N_PAD // NS
CROWS = N_PAD // D

BLK = 1000
GRID = N // BLK


def _make_stage(with_counts: bool):
    mesh = plsc.VectorSubcoreMesh(core_axis_name="c", subcore_axis_name="s")
    cp = pltpu.CompilerParams()
    if "needs_layout_passes" in pltpu.CompilerParams.__dataclass_fields__:
        cp = dataclasses.replace(cp, needs_layout_passes=False)
    out_type = [jax.ShapeDtypeStruct((NC * N_PAD, D), jnp.float32)]
    scratch = [
        pltpu.VMEM((CHUNK,), jnp.int32),
        pltpu.VMEM((CHUNK,), jnp.int32),
        pltpu.VMEM((CHUNK, D), jnp.float32),
        pltpu.VMEM_SHARED((N_PAD, D), jnp.float32),
        pltpu.SemaphoreType.DMA,
    ]
    if with_counts:
        out_type.append(jax.ShapeDtypeStruct((NC * CROWS, D), jnp.float32))
        scratch += [
            pltpu.VMEM((CROWS, D), jnp.float32),
            pltpu.VMEM((CROWS,), jnp.int32),
            pltpu.VMEM_SHARED((CROWS, D), jnp.float32),
        ]

    def body(*refs):
        if with_counts:
            (table, src_h, dst_h, zrows, out_acc, out_cnt,
             src_v, dst_v, rows_v, acc_sh, sem, cnt_v, idv, cnt_sh) = refs
        else:
            (table, src_h, dst_h, zrows, out_acc,
             src_v, dst_v, rows_v, acc_sh, sem) = refs
        c = lax.axis_index("c")
        s = lax.axis_index("s")
        wid = c * NS + s
        base = wid * PAIRS_PER_TILE
        row0 = s * ROWS_PER_TILE
        orow0 = c * N_PAD + row0

        pltpu.sync_copy(zrows.at[pl.ds(row0, ROWS_PER_TILE)],
                        acc_sh.at[pl.ds(row0, ROWS_PER_TILE)])
        if with_counts:
            pltpu.sync_copy(zrows.at[pl.ds(0, CROWS)], cnt_v)

            @pl.when(s == 0)
            def _():
                pltpu.sync_copy(zrows.at[pl.ds(0, CROWS)], cnt_sh)

            @pl.loop(0, CROWS // 16)
            def _(j):
                idv[pl.ds(j * 16, 16)] = lax.iota(jnp.int32, 16) + j * 16
        plsc.subcore_barrier()

        ones16 = jnp.ones((16,), jnp.float32)

        @pl.loop(0, N_ITERS)
        def _(it):
            off = base + it * CHUNK
            pltpu.sync_copy(src_h.at[pl.ds(off, CHUNK)], src_v)
            pltpu.sync_copy(dst_h.at[pl.ds(off, CHUNK)], dst_v)
            pltpu.async_copy(table.at[src_v], rows_v, sem).wait()
            pltpu.sync_copy(rows_v, acc_sh.at[dst_v], add=True)
            if with_counts:
                for j in range(CHUNK // 16):
                    v = dst_v[pl.ds(j * 16, 16)]
                    row = lax.shift_right_logical(v, 7)
                    lane = lax.bitwise_and(v, 127)
                    plsc.addupdate_scatter(cnt_v, [row, lane], ones16)

        plsc.subcore_barrier()
        if with_counts:
            pltpu.sync_copy(cnt_v, cnt_sh.at[idv], add=True)
            plsc.subcore_barrier()

            @pl.when(s == 0)
            def _():
                pltpu.sync_copy(cnt_sh, out_cnt.at[pl.ds(c * CROWS, CROWS)])
        pltpu.sync_copy(acc_sh.at[pl.ds(row0, ROWS_PER_TILE)],
                        out_acc.at[pl.ds(orow0, ROWS_PER_TILE)])

    return pl.kernel(body, out_type=tuple(out_type), mesh=mesh,
                     compiler_params=cp, scratch_types=scratch)


_make_stage = functools.cache(_make_stage)


def _row_spec():
    return pl.BlockSpec((BLK, D), lambda i: (i, 0))


def _w_spec():
    return pl.BlockSpec((D, D), lambda i: (0, 0))


def _b_spec():
    return pl.BlockSpec((1, D), lambda i: (0, 0))


def _p_spec():
    return pl.BlockSpec((NC, BLK, D), lambda i: (0, i, 0))


def _c_spec():
    return pl.BlockSpec((BLK, 1), lambda i: (i, 0))


def _pre_body(pd_r, wt1_r, bt1_r, wt2_r, bt2_r, x_r, wg0_r, bg0_r,
              wl0_r, bl0_r, topo_r, h0_r, t0_r):
    z = jnp.maximum(jnp.dot(pd_r[...], wt1_r[...],
                            preferred_element_type=jnp.float32) + bt1_r[...],
                    0.0)
    topo = jnp.dot(z, wt2_r[...], preferred_element_type=jnp.float32) + bt2_r[...]
    topo_r[...] = topo
    h0_r[...] = jnp.dot(x_r[...], wg0_r[...],
                        preferred_element_type=jnp.float32) + bg0_r[...]
    t0_r[...] = jnp.dot(topo, wl0_r[...],
                        preferred_element_type=jnp.float32) + bl0_r[...]


_k_pre = pl.pallas_call(
    _pre_body,
    grid=(GRID,),
    in_specs=[
        pl.BlockSpec((BLK, 8), lambda i: (i, 0)),
        pl.BlockSpec((8, D), lambda i: (0, 0)),
        _b_spec(), _w_spec(), _b_spec(),
        _row_spec(), _w_spec(), _b_spec(), _w_spec(), _b_spec(),
    ],
    out_specs=[_row_spec(), _row_spec(), _row_spec()],
    out_shape=[jax.ShapeDtypeStruct((N, D), jnp.float32)] * 3,
)


def _scale_body(p_r, c_r, o_r):
    y = p_r[0] + p_r[1]
    o_r[...] = y / jnp.maximum(c_r[...], 1.0)


_k_scale = pl.pallas_call(
    _scale_body,
    grid=(GRID,),
    in_specs=[_p_spec(), _c_spec()],
    out_specs=_row_spec(),
    out_shape=jax.ShapeDtypeStruct((N, D), jnp.float32),
)


def _mid_body(p_r, c_r, t0_r, wg1_r, bg1_r, wl1_r, bl1_r, h1_r, t1b_r):
    y = p_r[0] + p_r[1]
    y = y / jnp.maximum(c_r[...], 1.0)
    t0 = t0_r[...]
    x1 = jnp.maximum(y + y * t0, 0.0)
    h1_r[...] = jnp.dot(x1, wg1_r[...],
                        preferred_element_type=jnp.float32) + bg1_r[...]
    t1b_r[...] = jnp.dot(t0, wl1_r[...],
                         preferred_element_type=jnp.float32) + bl1_r[...]


_k_mid = pl.pallas_call(
    _mid_body,
    grid=(GRID,),
    in_specs=[_p_spec(), _c_spec(), _row_spec(),
              _w_spec(), _b_spec(), _w_spec(), _b_spec()],
    out_specs=[_row_spec(), _row_spec()],
    out_shape=[jax.ShapeDtypeStruct((N, D), jnp.float32)] * 2,
)


def _final_body(p_r, c_r, t_r, o_r):
    y = p_r[0] + p_r[1]
    y = y / jnp.maximum(c_r[...], 1.0)
    o_r[...] = jnp.maximum(y + y * t_r[...], 0.0)


_k_final = pl.pallas_call(
    _final_body,
    grid=(GRID,),
    in_specs=[_p_spec(), _c_spec(), _row_spec()],
    out_specs=_row_spec(),
    out_shape=jax.ShapeDtypeStruct((N, D), jnp.float32),
)


def _seg_sum_counts(table, src, dst, zrows):
    p, cnt = _make_stage(True)(table, src, dst, zrows)
    cnt2 = (cnt[:CROWS] + cnt[CROWS:]).reshape(N_PAD, 1)
    return p.reshape(NC, N_PAD, D), cnt2


def _seg_sum(table, src, dst, zrows):
    p, = _make_stage(False)(table, src, dst, zrows)
    return p.reshape(NC, N_PAD, D)


def kernel(x, hg, pd, Wt1, bt1, Wt2, bt2, Wg0, bg0, Wl0, bl0,
           Wg1, bg1, Wl1, bl1):
    node_idx = hg[0]
    edge_idx = hg[1]
    pd8 = jnp.pad(pd, ((0, 0), (0, 8 - pd.shape[1])))
    Wt1p = jnp.pad(Wt1, ((0, 8 - Wt1.shape[0]), (0, 0)))
    zrows = jnp.zeros((N_PAD, D), jnp.float32)
    b2 = lambda b: b.reshape(1, D)

    topo, h0, t0 = _k_pre(pd8, Wt1p, b2(bt1), Wt2, b2(bt2),
                          x, Wg0, b2(bg0), Wl0, b2(bl0))
    p1, ce = _seg_sum_counts(h0, node_idx, edge_idx, zrows)
    e_feat = _k_scale(p1, ce)
    p2, cn = _seg_sum_counts(e_feat, edge_idx, node_idx, zrows)
    h1, t1b = _k_mid(p2, cn, t0, Wg1, b2(bg1), Wl1, b2(bl1))
    p3 = _seg_sum(h1, node_idx, edge_idx, zrows)
    e_feat1 = _k_scale(p3, ce)
    p4 = _seg_sum(e_feat1, edge_idx, node_idx, zrows)
    out = _k_final(p4, cn, t1b)
    return (out, topo)


# speedup vs baseline: 8.5766x; 1.8777x over previous
import dataclasses
import functools

import jax
import jax.numpy as jnp
from jax import lax
from jax.experimental import pallas as pl
from jax.experimental.pallas import tpu as pltpu
from jax.experimental.pallas import tpu_sc as plsc

N = 10000
E = 320000
D = 128
CNT_W = 16
NC = 2
NS = 16
NW = NC * NS
PAIRS_PER_TILE = E // NW
CHUNK = 80
N_ITERS = PAIRS_PER_TILE // CHUNK
N_PAD = 10240
ROWS_PER_TILE = N_PAD // NS
CROWS = N_PAD // D

BLK = 1000
GRID = N // BLK


def _make_stage(with_counts: bool):
    mesh = plsc.VectorSubcoreMesh(core_axis_name="c", subcore_axis_name="s")
    cp = pltpu.CompilerParams()
    if "needs_layout_passes" in pltpu.CompilerParams.__dataclass_fields__:
        cp = dataclasses.replace(cp, needs_layout_passes=False)
    out_type = [jax.ShapeDtypeStruct((NC * N_PAD, D), jnp.float32)]
    scratch = [
        pltpu.VMEM((2, CHUNK), jnp.int32),
        pltpu.VMEM((2, CHUNK), jnp.int32),
        pltpu.VMEM((CHUNK, D), jnp.float32),
        pltpu.VMEM((CHUNK, D), jnp.float32),
        pltpu.VMEM_SHARED((N_PAD, D), jnp.float32),
        pltpu.SemaphoreType.DMA,
        pltpu.SemaphoreType.DMA,
    ]
    if with_counts:
        out_type.append(jax.ShapeDtypeStruct((NC * CROWS, D), jnp.float32))
        scratch += [
            pltpu.VMEM((CROWS, D), jnp.float32),
            pltpu.VMEM((CROWS,), jnp.int32),
            pltpu.VMEM_SHARED((CROWS, D), jnp.float32),
        ]

    def body(*refs):
        if with_counts:
            (table, pairs3, zrows, out_acc, out_cnt,
             pb0, pb1, rows0, rows1, acc_sh, sem0, sem1,
             cnt_v, idv, cnt_sh) = refs
        else:
            (table, pairs3, zrows, out_acc,
             pb0, pb1, rows0, rows1, acc_sh, sem0, sem1) = refs
        c = lax.axis_index("c")
        s = lax.axis_index("s")
        wid = c * NS + s
        pbase = wid * N_ITERS
        row0 = s * ROWS_PER_TILE
        orow0 = c * N_PAD + row0

        pltpu.sync_copy(zrows.at[pl.ds(row0, ROWS_PER_TILE)],
                        acc_sh.at[pl.ds(row0, ROWS_PER_TILE)])
        if with_counts:
            pltpu.sync_copy(zrows.at[pl.ds(0, CROWS)], cnt_v)

            @pl.when(s == 0)
            def _():
                pltpu.sync_copy(zrows.at[pl.ds(0, CROWS)], cnt_sh)

            @pl.loop(0, CROWS // 16)
            def _(j):
                idv[pl.ds(j * 16, 16)] = lax.iota(jnp.int32, 16) + j * 16
        plsc.subcore_barrier()

        ones16 = jnp.ones((16,), jnp.float32)

        def p_load(it, pb):
            pltpu.sync_copy(pairs3.at[pbase + it], pb)

        def g_start(pb, buf, sem):
            pltpu.async_copy(table.at[pb.at[0]], buf, sem)

        def g_wait(pb, buf, sem):
            pltpu.make_async_copy(table.at[pb.at[0]], buf, sem).wait()

        def scat(pb, buf):
            pltpu.sync_copy(buf, acc_sh.at[pb.at[1]], add=True)

        def count(pb):
            if with_counts:
                for j in range(CHUNK // 16):
                    v = pb[1, pl.ds(j * 16, 16)]
                    r = lax.shift_right_logical(v, 7)
                    l = lax.bitwise_and(v, 127)
                    plsc.addupdate_scatter(cnt_v, [r, l], ones16)

        p_load(0, pb0)
        g_start(pb0, rows0, sem0)

        @pl.loop(0, (N_ITERS - 1) // 2)
        def _(k):
            it0 = 2 * k
            p_load(it0 + 1, pb1)
            g_start(pb1, rows1, sem1)
            g_wait(pb0, rows0, sem0)
            scat(pb0, rows0)
            count(pb0)
            p_load(it0 + 2, pb0)
            g_start(pb0, rows0, sem0)
            g_wait(pb1, rows1, sem1)
            scat(pb1, rows1)
            count(pb1)

        g_wait(pb0, rows0, sem0)
        scat(pb0, rows0)
        count(pb0)

        plsc.subcore_barrier()
        if with_counts:
            pltpu.sync_copy(cnt_v, cnt_sh.at[idv], add=True)
            plsc.subcore_barrier()

            @pl.when(s == 0)
            def _():
                pltpu.sync_copy(cnt_sh, out_cnt.at[pl.ds(c * CROWS, CROWS)])
        pltpu.sync_copy(acc_sh.at[pl.ds(row0, ROWS_PER_TILE)],
                        out_acc.at[pl.ds(orow0, ROWS_PER_TILE)])

    return pl.kernel(body, out_type=tuple(out_type), mesh=mesh,
                     compiler_params=cp, scratch_types=scratch)


_make_stage = functools.cache(_make_stage)


def _row_spec():
    return pl.BlockSpec((BLK, D), lambda i: (i, 0))


def _w_spec():
    return pl.BlockSpec((D, D), lambda i: (0, 0))


def _b_spec():
    return pl.BlockSpec((1, D), lambda i: (0, 0))


def _p_spec():
    return pl.BlockSpec((NC, BLK, D), lambda i: (0, i, 0))


def _c_spec():
    return pl.BlockSpec((BLK, 1), lambda i: (i, 0))


def _pre_body(pd_r, wt1_r, bt1_r, wt2_r, bt2_r, x_r, wg0_r, bg0_r,
              wl0_r, bl0_r, topo_r, h0_r, t0_r):
    z = jnp.maximum(jnp.dot(pd_r[...], wt1_r[...],
                            preferred_element_type=jnp.float32) + bt1_r[...],
                    0.0)
    topo = jnp.dot(z, wt2_r[...], preferred_element_type=jnp.float32) + bt2_r[...]
    topo_r[...] = topo
    h0_r[...] = jnp.dot(x_r[...], wg0_r[...],
                        preferred_element_type=jnp.float32) + bg0_r[...]
    t0_r[...] = jnp.dot(topo, wl0_r[...],
                        preferred_element_type=jnp.float32) + bl0_r[...]


_k_pre = pl.pallas_call(
    _pre_body,
    grid=(GRID,),
    in_specs=[
        pl.BlockSpec((BLK, 8), lambda i: (i, 0)),
        pl.BlockSpec((8, D), lambda i: (0, 0)),
        _b_spec(), _w_spec(), _b_spec(),
        _row_spec(), _w_spec(), _b_spec(), _w_spec(), _b_spec(),
    ],
    out_specs=[_row_spec(), _row_spec(), _row_spec()],
    out_shape=[jax.ShapeDtypeStruct((N, D), jnp.float32)] * 3,
)


def _scale_body(p_r, c_r, o_r):
    y = p_r[0] + p_r[1]
    o_r[...] = y / jnp.maximum(c_r[...], 1.0)


_k_scale = pl.pallas_call(
    _scale_body,
    grid=(GRID,),
    in_specs=[_p_spec(), _c_spec()],
    out_specs=_row_spec(),
    out_shape=jax.ShapeDtypeStruct((N, D), jnp.float32),
)


def _mid_body(p_r, c_r, t0_r, wg1_r, bg1_r, wl1_r, bl1_r, h1_r, t1b_r):
    y = p_r[0] + p_r[1]
    y = y / jnp.maximum(c_r[...], 1.0)
    t0 = t0_r[...]
    x1 = jnp.maximum(y + y * t0, 0.0)
    h1_r[...] = jnp.dot(x1, wg1_r[...],
                        preferred_element_type=jnp.float32) + bg1_r[...]
    t1b_r[...] = jnp.dot(t0, wl1_r[...],
                         preferred_element_type=jnp.float32) + bl1_r[...]


_k_mid = pl.pallas_call(
    _mid_body,
    grid=(GRID,),
    in_specs=[_p_spec(), _c_spec(), _row_spec(),
              _w_spec(), _b_spec(), _w_spec(), _b_spec()],
    out_specs=[_row_spec(), _row_spec()],
    out_shape=[jax.ShapeDtypeStruct((N, D), jnp.float32)] * 2,
)


def _final_body(p_r, c_r, t_r, o_r):
    y = p_r[0] + p_r[1]
    y = y / jnp.maximum(c_r[...], 1.0)
    o_r[...] = jnp.maximum(y + y * t_r[...], 0.0)


_k_final = pl.pallas_call(
    _final_body,
    grid=(GRID,),
    in_specs=[_p_spec(), _c_spec(), _row_spec()],
    out_specs=_row_spec(),
    out_shape=jax.ShapeDtypeStruct((N, D), jnp.float32),
)


def _pairs3(src, dst):
    return jnp.stack([src.reshape(NW * N_ITERS, CHUNK),
                      dst.reshape(NW * N_ITERS, CHUNK)], axis=1)


def _seg_sum_counts(table, src, dst, zrows):
    p, cnt = _make_stage(True)(table, _pairs3(src, dst), zrows)
    cnt2 = (cnt[:CROWS] + cnt[CROWS:]).reshape(N_PAD, 1)
    return p.reshape(NC, N_PAD, D), cnt2


def _seg_sum(table, src, dst, zrows):
    p, = _make_stage(False)(table, _pairs3(src, dst), zrows)
    return p.reshape(NC, N_PAD, D)


def kernel(x, hg, pd, Wt1, bt1, Wt2, bt2, Wg0, bg0, Wl0, bl0,
           Wg1, bg1, Wl1, bl1):
    node_idx = hg[0]
    edge_idx = hg[1]
    pd8 = jnp.pad(pd, ((0, 0), (0, 8 - pd.shape[1])))
    Wt1p = jnp.pad(Wt1, ((0, 8 - Wt1.shape[0]), (0, 0)))
    zrows = jnp.zeros((N_PAD, D), jnp.float32)
    b2 = lambda b: b.reshape(1, D)

    topo, h0, t0 = _k_pre(pd8, Wt1p, b2(bt1), Wt2, b2(bt2),
                          x, Wg0, b2(bg0), Wl0, b2(bl0))
    p1, ce = _seg_sum_counts(h0, node_idx, edge_idx, zrows)
    e_feat = _k_scale(p1, ce)
    p2, cn = _seg_sum_counts(e_feat, edge_idx, node_idx, zrows)
    h1, t1b = _k_mid(p2, cn, t0, Wg1, b2(bg1), Wl1, b2(bl1))
    p3 = _seg_sum(h1, node_idx, edge_idx, zrows)
    e_feat1 = _k_scale(p3, ce)
    p4 = _seg_sum(e_feat1, edge_idx, node_idx, zrows)
    out = _k_final(p4, cn, t1b)
    return (out, topo)


# speedup vs baseline: 9.8049x; 1.1432x over previous
import dataclasses
import functools

import jax
import jax.numpy as jnp
from jax import lax
from jax.experimental import pallas as pl
from jax.experimental.pallas import tpu as pltpu
from jax.experimental.pallas import tpu_sc as plsc

N = 10000
E = 320000
D = 128
CNT_W = 16
NC = 2
NS = 16
NW = NC * NS
PAIRS_PER_TILE = E // NW
CHUNK = 80
N_ITERS = PAIRS_PER_TILE // CHUNK
N_PAD = 10240
ROWS_PER_TILE = N_PAD // NS
CROWS = N_PAD // D

BLK = 1000
GRID = N // BLK


def _make_stage(with_counts: bool):
    mesh = plsc.VectorSubcoreMesh(core_axis_name="c", subcore_axis_name="s")
    cp = pltpu.CompilerParams()
    if "needs_layout_passes" in pltpu.CompilerParams.__dataclass_fields__:
        cp = dataclasses.replace(cp, needs_layout_passes=False)
    out_type = [jax.ShapeDtypeStruct((NC * N_PAD, D), jnp.float32)]
    scratch = [
        pltpu.VMEM((2, CHUNK), jnp.int32),
        pltpu.VMEM((2, CHUNK), jnp.int32),
        pltpu.VMEM((2, CHUNK), jnp.int32),
        pltpu.VMEM((CHUNK, D), jnp.float32),
        pltpu.VMEM((CHUNK, D), jnp.float32),
        pltpu.VMEM((CHUNK, D), jnp.float32),
        pltpu.VMEM_SHARED((N_PAD, D), jnp.float32),
        pltpu.SemaphoreType.DMA,
        pltpu.SemaphoreType.DMA,
        pltpu.SemaphoreType.DMA,
        pltpu.SemaphoreType.DMA,
        pltpu.SemaphoreType.DMA,
        pltpu.SemaphoreType.DMA,
    ]
    if with_counts:
        out_type.append(jax.ShapeDtypeStruct((NC * CROWS, D), jnp.float32))
        scratch += [
            pltpu.VMEM((CROWS, D), jnp.float32),
            pltpu.VMEM((CROWS,), jnp.int32),
            pltpu.VMEM_SHARED((CROWS, D), jnp.float32),
        ]

    def body(*refs):
        if with_counts:
            (table, pairs3, zrows, out_acc, out_cnt,
             pb0, pb1, pb2, r0, r1, r2, acc_sh, g0, g1, g2, s0, s1, s2,
             cnt_v, idv, cnt_sh) = refs
        else:
            (table, pairs3, zrows, out_acc,
             pb0, pb1, pb2, r0, r1, r2, acc_sh, g0, g1, g2, s0, s1, s2) = refs
        PB, RW = (pb0, pb1, pb2), (r0, r1, r2)
        GS, SS = (g0, g1, g2), (s0, s1, s2)
        c = lax.axis_index("c")
        s = lax.axis_index("s")
        wid = c * NS + s
        pbase = wid * N_ITERS
        row0 = s * ROWS_PER_TILE
        orow0 = c * N_PAD + row0

        pltpu.sync_copy(zrows.at[pl.ds(row0, ROWS_PER_TILE)],
                        acc_sh.at[pl.ds(row0, ROWS_PER_TILE)])
        if with_counts:
            pltpu.sync_copy(zrows.at[pl.ds(0, CROWS)], cnt_v)

            @pl.when(s == 0)
            def _():
                pltpu.sync_copy(zrows.at[pl.ds(0, CROWS)], cnt_sh)

            @pl.loop(0, CROWS // 16)
            def _(j):
                idv[pl.ds(j * 16, 16)] = lax.iota(jnp.int32, 16) + j * 16
        plsc.subcore_barrier()

        ones16 = jnp.ones((16,), jnp.float32)

        def p_load(it, b):
            pltpu.sync_copy(pairs3.at[pbase + it], PB[b])

        def g_start(b):
            pltpu.async_copy(table.at[PB[b].at[0]], RW[b], GS[b])

        def g_wait(b):
            pltpu.make_async_copy(table.at[PB[b].at[0]], RW[b], GS[b]).wait()

        def s_start(b):
            pltpu.async_copy(RW[b], acc_sh.at[PB[b].at[1]], SS[b], add=True)

        def s_wait(b):
            pltpu.make_async_copy(RW[b], acc_sh.at[PB[b].at[1]], SS[b]).wait()

        def count(b):
            if with_counts:
                for j in range(CHUNK // 16):
                    v = PB[b][1, pl.ds(j * 16, 16)]
                    r = lax.shift_right_logical(v, 7)
                    l = lax.bitwise_and(v, 127)
                    plsc.addupdate_scatter(cnt_v, [r, l], ones16)

        p_load(0, 0)
        g_start(0)
        p_load(1, 1)
        g_start(1)
        g_wait(0)
        s_start(0)
        count(0)
        p_load(2, 2)
        g_start(2)
        g_wait(1)
        s_start(1)
        count(1)

        @pl.loop(1, (N_ITERS - 5) // 3 + 1)
        def _(k):
            it0 = 3 * k
            for j in range(3):
                s_wait(j)
                p_load(it0 + j, j)
                g_start(j)
                g_wait((j + 2) % 3)
                s_start((j + 2) % 3)
                count((j + 2) % 3)

        for t, j in ((N_ITERS - 2, 0), (N_ITERS - 1, 1)):
            s_wait(j)
            p_load(t, j)
            g_start(j)
            g_wait((j + 2) % 3)
            s_start((j + 2) % 3)
            count((j + 2) % 3)
        g_wait(1)
        s_start(1)
        count(1)
        s_wait(2)
        s_wait(0)
        s_wait(1)

        plsc.subcore_barrier()
        if with_counts:
            pltpu.sync_copy(cnt_v, cnt_sh.at[idv], add=True)
            plsc.subcore_barrier()

            @pl.when(s == 0)
            def _():
                pltpu.sync_copy(cnt_sh, out_cnt.at[pl.ds(c * CROWS, CROWS)])
        pltpu.sync_copy(acc_sh.at[pl.ds(row0, ROWS_PER_TILE)],
                        out_acc.at[pl.ds(orow0, ROWS_PER_TILE)])

    return pl.kernel(body, out_type=tuple(out_type), mesh=mesh,
                     compiler_params=cp, scratch_types=scratch)


_make_stage = functools.cache(_make_stage)


def _row_spec():
    return pl.BlockSpec((BLK, D), lambda i: (i, 0))


def _w_spec():
    return pl.BlockSpec((D, D), lambda i: (0, 0))


def _b_spec():
    return pl.BlockSpec((1, D), lambda i: (0, 0))


def _p_spec():
    return pl.BlockSpec((NC, BLK, D), lambda i: (0, i, 0))


def _c_spec():
    return pl.BlockSpec((BLK, 1), lambda i: (i, 0))


def _pre_body(pd_r, wt1_r, bt1_r, wt2_r, bt2_r, x_r, wg0_r, bg0_r,
              wl0_r, bl0_r, topo_r, h0_r, t0_r):
    z = jnp.maximum(jnp.dot(pd_r[...], wt1_r[...],
                            preferred_element_type=jnp.float32) + bt1_r[...],
                    0.0)
    topo = jnp.dot(z, wt2_r[...], preferred_element_type=jnp.float32) + bt2_r[...]
    topo_r[...] = topo
    h0_r[...] = jnp.dot(x_r[...], wg0_r[...],
                        preferred_element_type=jnp.float32) + bg0_r[...]
    t0_r[...] = jnp.dot(topo, wl0_r[...],
                        preferred_element_type=jnp.float32) + bl0_r[...]


_k_pre = pl.pallas_call(
    _pre_body,
    grid=(GRID,),
    in_specs=[
        pl.BlockSpec((BLK, 8), lambda i: (i, 0)),
        pl.BlockSpec((8, D), lambda i: (0, 0)),
        _b_spec(), _w_spec(), _b_spec(),
        _row_spec(), _w_spec(), _b_spec(), _w_spec(), _b_spec(),
    ],
    out_specs=[_row_spec(), _row_spec(), _row_spec()],
    out_shape=[jax.ShapeDtypeStruct((N, D), jnp.float32)] * 3,
)


def _scale_body(p_r, c_r, o_r):
    y = p_r[0] + p_r[1]
    o_r[...] = y / jnp.maximum(c_r[...], 1.0)


_k_scale = pl.pallas_call(
    _scale_body,
    grid=(GRID,),
    in_specs=[_p_spec(), _c_spec()],
    out_specs=_row_spec(),
    out_shape=jax.ShapeDtypeStruct((N, D), jnp.float32),
)


def _mid_body(p_r, c_r, t0_r, wg1_r, bg1_r, wl1_r, bl1_r, h1_r, t1b_r):
    y = p_r[0] + p_r[1]
    y = y / jnp.maximum(c_r[...], 1.0)
    t0 = t0_r[...]
    x1 = jnp.maximum(y + y * t0, 0.0)
    h1_r[...] = jnp.dot(x1, wg1_r[...],
                        preferred_element_type=jnp.float32) + bg1_r[...]
    t1b_r[...] = jnp.dot(t0, wl1_r[...],
                         preferred_element_type=jnp.float32) + bl1_r[...]


_k_mid = pl.pallas_call(
    _mid_body,
    grid=(GRID,),
    in_specs=[_p_spec(), _c_spec(), _row_spec(),
              _w_spec(), _b_spec(), _w_spec(), _b_spec()],
    out_specs=[_row_spec(), _row_spec()],
    out_shape=[jax.ShapeDtypeStruct((N, D), jnp.float32)] * 2,
)


def _final_body(p_r, c_r, t_r, o_r):
    y = p_r[0] + p_r[1]
    y = y / jnp.maximum(c_r[...], 1.0)
    o_r[...] = jnp.maximum(y + y * t_r[...], 0.0)


_k_final = pl.pallas_call(
    _final_body,
    grid=(GRID,),
    in_specs=[_p_spec(), _c_spec(), _row_spec()],
    out_specs=_row_spec(),
    out_shape=jax.ShapeDtypeStruct((N, D), jnp.float32),
)


def _pairs3(src, dst):
    return jnp.stack([src.reshape(NW * N_ITERS, CHUNK),
                      dst.reshape(NW * N_ITERS, CHUNK)], axis=1)


def _seg_sum_counts(table, src, dst, zrows):
    p, cnt = _make_stage(True)(table, _pairs3(src, dst), zrows)
    cnt2 = (cnt[:CROWS] + cnt[CROWS:]).reshape(N_PAD, 1)
    return p.reshape(NC, N_PAD, D), cnt2


def _seg_sum(table, src, dst, zrows):
    p, = _make_stage(False)(table, _pairs3(src, dst), zrows)
    return p.reshape(NC, N_PAD, D)


def kernel(x, hg, pd, Wt1, bt1, Wt2, bt2, Wg0, bg0, Wl0, bl0,
           Wg1, bg1, Wl1, bl1):
    node_idx = hg[0]
    edge_idx = hg[1]
    pd8 = jnp.pad(pd, ((0, 0), (0, 8 - pd.shape[1])))
    Wt1p = jnp.pad(Wt1, ((0, 8 - Wt1.shape[0]), (0, 0)))
    zrows = jnp.zeros((N_PAD, D), jnp.float32)
    b2 = lambda b: b.reshape(1, D)

    topo, h0, t0 = _k_pre(pd8, Wt1p, b2(bt1), Wt2, b2(bt2),
                          x, Wg0, b2(bg0), Wl0, b2(bl0))
    p1, ce = _seg_sum_counts(h0, node_idx, edge_idx, zrows)
    e_feat = _k_scale(p1, ce)
    p2, cn = _seg_sum_counts(e_feat, edge_idx, node_idx, zrows)
    h1, t1b = _k_mid(p2, cn, t0, Wg1, b2(bg1), Wl1, b2(bl1))
    p3 = _seg_sum(h1, node_idx, edge_idx, zrows)
    e_feat1 = _k_scale(p3, ce)
    p4 = _seg_sum(e_feat1, edge_idx, node_idx, zrows)
    out = _k_final(p4, cn, t1b)
    return (out, topo)


# speedup vs baseline: 11.6675x; 1.1900x over previous
import dataclasses
import functools

import jax
import jax.numpy as jnp
from jax import lax
from jax.experimental import pallas as pl
from jax.experimental.pallas import tpu as pltpu
from jax.experimental.pallas import tpu_sc as plsc

N = 10000
E = 320000
D = 128
CNT_W = 16
NC = 2
NS = 16
NW = NC * NS
PAIRS_PER_TILE = E // NW
CHUNK = 80
N_ITERS = PAIRS_PER_TILE // CHUNK
N_PAD = 10240
ROWS_PER_TILE = N_PAD // NS
CROWS = N_PAD // D

BLK = 1000
GRID = N // BLK


def _make_stage(with_counts: bool):
    mesh = plsc.VectorSubcoreMesh(core_axis_name="c", subcore_axis_name="s")
    cp = pltpu.CompilerParams()
    if "needs_layout_passes" in pltpu.CompilerParams.__dataclass_fields__:
        cp = dataclasses.replace(cp, needs_layout_passes=False)
    out_type = [jax.ShapeDtypeStruct((NC * N_PAD, D), jnp.float32)]
    scratch = (
        [pltpu.VMEM((2, CHUNK), jnp.int32)] * 6
        + [pltpu.VMEM((CHUNK, D), jnp.float32)] * 3
        + [pltpu.VMEM_SHARED((N_PAD, D), jnp.float32)]
        + [pltpu.SemaphoreType.DMA] * 12
    )
    if with_counts:
        out_type.append(jax.ShapeDtypeStruct((NC * CROWS, D), jnp.float32))
        scratch += [
            pltpu.VMEM((CROWS, D), jnp.float32),
            pltpu.VMEM((CROWS,), jnp.int32),
            pltpu.VMEM_SHARED((CROWS, D), jnp.float32),
        ]

    def body(*refs):
        if with_counts:
            (table, pairs3, zrows, out_acc, out_cnt), rest = refs[:5], refs[5:]
            cnt_v, idv, cnt_sh = rest[22:25]
        else:
            (table, pairs3, zrows, out_acc), rest = refs[:4], refs[4:]
        PB = rest[0:6]
        RW = rest[6:9]
        acc_sh = rest[9]
        GS = rest[10:13]
        SS = rest[13:16]
        PS = rest[16:22]
        c = lax.axis_index("c")
        s = lax.axis_index("s")
        wid = c * NS + s
        pbase = wid * N_ITERS
        row0 = s * ROWS_PER_TILE
        orow0 = c * N_PAD + row0

        pltpu.sync_copy(zrows.at[pl.ds(row0, ROWS_PER_TILE)],
                        acc_sh.at[pl.ds(row0, ROWS_PER_TILE)])
        if with_counts:
            pltpu.sync_copy(zrows.at[pl.ds(0, CROWS)], cnt_v)

            @pl.when(s == 0)
            def _():
                pltpu.sync_copy(zrows.at[pl.ds(0, CROWS)], cnt_sh)

            @pl.loop(0, CROWS // 16)
            def _(j):
                idv[pl.ds(j * 16, 16)] = lax.iota(jnp.int32, 16) + j * 16
        plsc.subcore_barrier()

        ones16 = jnp.ones((16,), jnp.float32)

        def p_start(it, q):
            pltpu.async_copy(pairs3.at[pbase + it], PB[q], PS[q])

        def p_wait(it, q):
            pltpu.make_async_copy(pairs3.at[pbase + it], PB[q], PS[q]).wait()

        def g_start(b, q):
            pltpu.async_copy(table.at[PB[q].at[0]], RW[b], GS[b])

        def g_wait(b, q):
            pltpu.make_async_copy(table.at[PB[q].at[0]], RW[b], GS[b]).wait()

        def s_start(b, q):
            pltpu.async_copy(RW[b], acc_sh.at[PB[q].at[1]], SS[b], add=True)

        def s_wait(b, q):
            pltpu.make_async_copy(RW[b], acc_sh.at[PB[q].at[1]], SS[b]).wait()

        def count(q):
            if with_counts:
                for j in range(CHUNK // 16):
                    v = PB[q][1, pl.ds(j * 16, 16)]
                    r = lax.shift_right_logical(v, 7)
                    l = lax.bitwise_and(v, 127)
                    plsc.addupdate_scatter(cnt_v, [r, l], ones16)

        def slot(it, j, head=False, tail_pstart=True, has_prev=True):
            b, q = j % 3, j % 6
            qn = (j + 3) % 6
            if not head:
                s_wait(b, qn)
            if tail_pstart:
                p_start(it + 3, qn)
            p_wait(it, q)
            g_start(b, q)
            if has_prev:
                bp, qp = (j + 2) % 3, (j + 5) % 6
                g_wait(bp, qp)
                s_start(bp, qp)
                count(qp)

        for it in range(3):
            p_start(it, it)
        for it in range(3):
            slot(it, it, head=True, has_prev=(it > 0))
        for it in range(3, 6):
            slot(it, it)

        @pl.loop(1, (N_ITERS - 11) // 6 + 1)
        def _(k):
            it0 = 6 * k
            for j in range(6):
                slot(it0 + j, j)

        for it in range(N_ITERS - 5, N_ITERS):
            slot(it, it % 6, tail_pstart=(it + 3 < N_ITERS))
        bl, ql = (N_ITERS - 1) % 3, (N_ITERS - 1) % 6
        g_wait(bl, ql)
        s_start(bl, ql)
        count(ql)
        for d in range(N_ITERS - 3, N_ITERS):
            s_wait(d % 3, d % 6)

        plsc.subcore_barrier()
        if with_counts:
            pltpu.sync_copy(cnt_v, cnt_sh.at[idv], add=True)
            plsc.subcore_barrier()

            @pl.when(s == 0)
            def _():
                pltpu.sync_copy(cnt_sh, out_cnt.at[pl.ds(c * CROWS, CROWS)])
        pltpu.sync_copy(acc_sh.at[pl.ds(row0, ROWS_PER_TILE)],
                        out_acc.at[pl.ds(orow0, ROWS_PER_TILE)])

    return pl.kernel(body, out_type=tuple(out_type), mesh=mesh,
                     compiler_params=cp, scratch_types=scratch)


_make_stage = functools.cache(_make_stage)


def _row_spec():
    return pl.BlockSpec((BLK, D), lambda i: (i, 0))


def _w_spec():
    return pl.BlockSpec((D, D), lambda i: (0, 0))


def _b_spec():
    return pl.BlockSpec((1, D), lambda i: (0, 0))


def _p_spec():
    return pl.BlockSpec((NC, BLK, D), lambda i: (0, i, 0))


def _c_spec():
    return pl.BlockSpec((BLK, 1), lambda i: (i, 0))


def _pre_body(pd_r, wt1_r, bt1_r, wt2_r, bt2_r, x_r, wg0_r, bg0_r,
              wl0_r, bl0_r, topo_r, h0_r, t0_r):
    z = jnp.maximum(jnp.dot(pd_r[...], wt1_r[...],
                            preferred_element_type=jnp.float32) + bt1_r[...],
                    0.0)
    topo = jnp.dot(z, wt2_r[...], preferred_element_type=jnp.float32) + bt2_r[...]
    topo_r[...] = topo
    h0_r[...] = jnp.dot(x_r[...], wg0_r[...],
                        preferred_element_type=jnp.float32) + bg0_r[...]
    t0_r[...] = jnp.dot(topo, wl0_r[...],
                        preferred_element_type=jnp.float32) + bl0_r[...]


_k_pre = pl.pallas_call(
    _pre_body,
    grid=(GRID,),
    in_specs=[
        pl.BlockSpec((BLK, 8), lambda i: (i, 0)),
        pl.BlockSpec((8, D), lambda i: (0, 0)),
        _b_spec(), _w_spec(), _b_spec(),
        _row_spec(), _w_spec(), _b_spec(), _w_spec(), _b_spec(),
    ],
    out_specs=[_row_spec(), _row_spec(), _row_spec()],
    out_shape=[jax.ShapeDtypeStruct((N, D), jnp.float32)] * 3,
)


def _scale_body(p_r, c_r, o_r):
    y = p_r[0] + p_r[1]
    o_r[...] = y / jnp.maximum(c_r[...], 1.0)


_k_scale = pl.pallas_call(
    _scale_body,
    grid=(GRID,),
    in_specs=[_p_spec(), _c_spec()],
    out_specs=_row_spec(),
    out_shape=jax.ShapeDtypeStruct((N, D), jnp.float32),
)


def _mid_body(p_r, c_r, t0_r, wg1_r, bg1_r, wl1_r, bl1_r, h1_r, t1b_r):
    y = p_r[0] + p_r[1]
    y = y / jnp.maximum(c_r[...], 1.0)
    t0 = t0_r[...]
    x1 = jnp.maximum(y + y * t0, 0.0)
    h1_r[...] = jnp.dot(x1, wg1_r[...],
                        preferred_element_type=jnp.float32) + bg1_r[...]
    t1b_r[...] = jnp.dot(t0, wl1_r[...],
                         preferred_element_type=jnp.float32) + bl1_r[...]


_k_mid = pl.pallas_call(
    _mid_body,
    grid=(GRID,),
    in_specs=[_p_spec(), _c_spec(), _row_spec(),
              _w_spec(), _b_spec(), _w_spec(), _b_spec()],
    out_specs=[_row_spec(), _row_spec()],
    out_shape=[jax.ShapeDtypeStruct((N, D), jnp.float32)] * 2,
)


def _final_body(p_r, c_r, t_r, o_r):
    y = p_r[0] + p_r[1]
    y = y / jnp.maximum(c_r[...], 1.0)
    o_r[...] = jnp.maximum(y + y * t_r[...], 0.0)


_k_final = pl.pallas_call(
    _final_body,
    grid=(GRID,),
    in_specs=[_p_spec(), _c_spec(), _row_spec()],
    out_specs=_row_spec(),
    out_shape=jax.ShapeDtypeStruct((N, D), jnp.float32),
)


def _pairs3(src, dst):
    return jnp.stack([src.reshape(NW * N_ITERS, CHUNK),
                      dst.reshape(NW * N_ITERS, CHUNK)], axis=1)


def _seg_sum_counts(table, src, dst, zrows):
    p, cnt = _make_stage(True)(table, _pairs3(src, dst), zrows)
    cnt2 = (cnt[:CROWS] + cnt[CROWS:]).reshape(N_PAD, 1)
    return p.reshape(NC, N_PAD, D), cnt2


def _seg_sum(table, src, dst, zrows):
    p, = _make_stage(False)(table, _pairs3(src, dst), zrows)
    return p.reshape(NC, N_PAD, D)


def kernel(x, hg, pd, Wt1, bt1, Wt2, bt2, Wg0, bg0, Wl0, bl0,
           Wg1, bg1, Wl1, bl1):
    node_idx = hg[0]
    edge_idx = hg[1]
    pd8 = jnp.pad(pd, ((0, 0), (0, 8 - pd.shape[1])))
    Wt1p = jnp.pad(Wt1, ((0, 8 - Wt1.shape[0]), (0, 0)))
    zrows = jnp.zeros((N_PAD, D), jnp.float32)
    b2 = lambda b: b.reshape(1, D)

    topo, h0, t0 = _k_pre(pd8, Wt1p, b2(bt1), Wt2, b2(bt2),
                          x, Wg0, b2(bg0), Wl0, b2(bl0))
    p1, ce = _seg_sum_counts(h0, node_idx, edge_idx, zrows)
    e_feat = _k_scale(p1, ce)
    p2, cn = _seg_sum_counts(e_feat, edge_idx, node_idx, zrows)
    h1, t1b = _k_mid(p2, cn, t0, Wg1, b2(bg1), Wl1, b2(bl1))
    p3 = _seg_sum(h1, node_idx, edge_idx, zrows)
    e_feat1 = _k_scale(p3, ce)
    p4 = _seg_sum(e_feat1, edge_idx, node_idx, zrows)
    out = _k_final(p4, cn, t1b)
    return (out, topo)
